# Initial kernel scaffold; baseline (speedup 1.0000x reference)
import jax
import jax.numpy as jnp
from jax.experimental import pallas as pl


def kernel(x, z, params):
    raise NotImplementedError("write your pallas kernel here")


# speedup vs baseline: 5.6315x; 5.6315x over previous
import functools

import jax
import jax.numpy as jnp
from jax import lax
from jax.experimental import pallas as pl
from jax.experimental.pallas import tpu as pltpu

B, N = 2, 1024
DIM, HIDDEN = 128, 4
KNN, M_DIM = 32, 16
EDGE_IN = 2 * DIM + 1
ROWS = 256


def _silu(t):
    return t * (1.0 / (1.0 + jnp.exp(-t)))


def _ln(x, g, b):
    mu = jnp.mean(x, axis=-1, keepdims=True)
    var = jnp.mean((x - mu) ** 2, axis=-1, keepdims=True)
    return (x - mu) / jnp.sqrt(var + 1e-5) * g + b


def _dot(a, b):
    return jnp.dot(a, b, preferred_element_type=jnp.float32)


def _embed_body(z_ref, tok_ref, pos_ref, out_ref):
    z = z_ref[0]
    num_tok = tok_ref.shape[0]
    cols = lax.broadcasted_iota(jnp.int32, (N, num_tok), 1)
    onehot = (z == cols).astype(jnp.float32)
    emb = jnp.dot(onehot, tok_ref[...], precision=lax.Precision.HIGHEST,
                  preferred_element_type=jnp.float32)
    out_ref[0] = emb + pos_ref[...]


def _embed(z, token_emb, pos_emb):
    z2 = z.reshape(B, N, 1).astype(jnp.int32)
    return pl.pallas_call(
        _embed_body,
        grid=(B,),
        in_specs=[
            pl.BlockSpec((1, N, 1), lambda b: (b, 0, 0)),
            pl.BlockSpec(token_emb.shape, lambda b: (0, 0)),
            pl.BlockSpec(pos_emb.shape, lambda b: (0, 0)),
        ],
        out_specs=pl.BlockSpec((1, N, DIM), lambda b: (b, 0, 0)),
        out_shape=jax.ShapeDtypeStruct((B, N, DIM), jnp.float32),
    )(z2, token_emb, pos_emb)


def _layer_body(
    feats_all_ref, feats_blk_ref, coorsT_ref, coors_blk_ref,
    w1a_ref, w1b_ref, w1d_ref, b1_ref, w2_ref, b2_ref,
    cw1_ref, cb1_ref, cw2_ref, cb2_ref, cns_ref,
    nng_ref, nnb_ref, nw1a_ref, nw1b_ref, nb1_ref, nw2_ref, nb2_ref,
    feats_out_ref, coors_out_ref, d_ref,
):
    feats_all = feats_all_ref[0]
    feats_blk = feats_blk_ref[0]
    coors_blk = coors_blk_ref[0]

    d = None
    for c in range(3):
        xi_c = coors_blk[:, c:c + 1]
        xj_c = coorsT_ref[0, c:c + 1, :]
        rel_c = xi_c - xj_c
        sq = rel_c * rel_c
        d = sq if d is None else d + sq
    d_ref[...] = d

    gi = _dot(feats_blk, w1a_ref[...]) + b1_ref[...]

    cols = lax.broadcasted_iota(jnp.int32, (ROWS, N), 1)

    def body(k, carry):
        m_i, cd0_, cd1_, cd2_ = carry
        dcur = d_ref[...]
        mval = jnp.min(dcur, axis=1, keepdims=True)
        jstar = jnp.min(jnp.where(dcur == mval, cols, N),
                        axis=1, keepdims=True)
        hot = cols == jstar
        d_ref[...] = jnp.where(hot, jnp.inf, dcur)
        onehot = hot.astype(jnp.float32)

        fj = _dot(onehot, feats_all)
        xjs = [jnp.sum(jnp.where(hot, coorsT_ref[0, c:c + 1, :], 0.0),
                       axis=1, keepdims=True) for c in range(3)]
        relk = [coors_blk[:, c:c + 1] - xjs[c] for c in range(3)]

        pre = _dot(fj, w1b_ref[...]) + gi + mval * w1d_ref[...]
        h = _silu(pre)
        mk = _silu(_dot(h, w2_ref[...]) + b2_ref[...])
        cwh = _silu(_dot(mk, cw1_ref[...]) + cb1_ref[...])
        cw = _dot(cwh, cw2_ref[...]) + cb2_ref[...]

        scale = cw / jnp.clip(jnp.sqrt(mval), 1e-8) * cns_ref[0, 0]
        return (m_i + mk, cd0_ + scale * relk[0],
                cd1_ + scale * relk[1], cd2_ + scale * relk[2])

    zero1 = jnp.zeros((ROWS, 1), jnp.float32)
    m_i, cd0, cd1, cd2 = lax.fori_loop(
        0, KNN, body, (jnp.zeros((ROWS, M_DIM), jnp.float32), zero1, zero1, zero1))
    cdelta = jnp.concatenate([cd0, cd1, cd2], axis=1)

    nf = _ln(feats_blk, nng_ref[...], nnb_ref[...])
    pre2 = _dot(nf, nw1a_ref[...]) + _dot(m_i, nw1b_ref[...]) + nb1_ref[...]
    node_out = _dot(_silu(pre2), nw2_ref[...]) + nb2_ref[...] + feats_blk

    feats_out_ref[0] = node_out
    coors_out_ref[0] = coors_blk + cdelta


def _layer(feats, coors, lp):
    coorsT = jnp.swapaxes(coors, 1, 2)
    w1a = lp['e_W1'][:DIM]
    w1b = lp['e_W1'][DIM:2 * DIM]
    w1d = lp['e_W1'][2 * DIM:2 * DIM + 1]
    nw1a = lp['n_W1'][:DIM]
    nw1b = lp['n_W1'][DIM:]
    row2 = lambda v: v.reshape(1, -1)

    grid = (B, N // ROWS)
    full = lambda shape: pl.BlockSpec(shape, lambda b, r: (0,) * len(shape))
    perb = lambda shape: pl.BlockSpec((1,) + shape, lambda b, r: (b,) + (0,) * len(shape))
    blk = lambda shape: pl.BlockSpec((1, ROWS) + shape, lambda b, r: (b, r) + (0,) * len(shape))

    out = pl.pallas_call(
        _layer_body,
        grid=grid,
        in_specs=[
            perb((N, DIM)),
            blk((DIM,)),
            perb((3, N)),
            blk((3,)),
            full(w1a.shape), full(w1b.shape), full((1, 2 * EDGE_IN)),
            full((1, 2 * EDGE_IN)),
            full(lp['e_W2'].shape), full((1, M_DIM)),
            full(lp['c_W1'].shape), full((1, 4 * M_DIM)),
            full(lp['c_W2'].shape), full((1, 1)), full((1, 1)),
            full((1, DIM)), full((1, DIM)),
            full(nw1a.shape), full(nw1b.shape), full((1, 2 * DIM)),
            full(lp['n_W2'].shape), full((1, DIM)),
        ],
        out_specs=[blk((DIM,)), blk((3,))],
        out_shape=[
            jax.ShapeDtypeStruct((B, N, DIM), jnp.float32),
            jax.ShapeDtypeStruct((B, N, 3), jnp.float32),
        ],
        scratch_shapes=[pltpu.VMEM((ROWS, N), jnp.float32)],
    )(
        feats, feats, coorsT, coors,
        w1a, w1b, row2(w1d), row2(lp['e_b1']), lp['e_W2'], row2(lp['e_b2']),
        lp['c_W1'], row2(lp['c_b1']), lp['c_W2'], row2(lp['c_b2']),
        row2(lp['cn_scale']),
        row2(lp['nn_g']), row2(lp['nn_b']),
        nw1a, nw1b, row2(lp['n_b1']), lp['n_W2'], row2(lp['n_b2']),
    )
    return out[0], out[1]


def _head_body(x_ref, w1_ref, b1_ref, w2_ref, b2_ref, pa_ref,
               g1_ref, bb1_ref, g2_ref, bb2_ref, out_ref):
    x = x_ref[0]
    h = _ln(x + x, g1_ref[...], bb1_ref[...])
    a = _dot(h, w1_ref[...]) + b1_ref[...]
    a = jnp.where(a >= 0, a, pa_ref[0, 0] * a)
    h2 = _dot(a, w2_ref[...]) + b2_ref[...]
    out_ref[0] = _ln(h + h2, g2_ref[...], bb2_ref[...])


def _head(feats, params):
    row2 = lambda v: v.reshape(1, -1)
    full = lambda shape: pl.BlockSpec(shape, lambda b: (0,) * len(shape))
    return pl.pallas_call(
        _head_body,
        grid=(B,),
        in_specs=[
            pl.BlockSpec((1, N, DIM), lambda b: (b, 0, 0)),
            full(params['ffn_W1'].shape), full((1, HIDDEN * DIM)),
            full(params['ffn_W2'].shape), full((1, DIM)),
            full((1, 1)),
            full((1, DIM)), full((1, DIM)), full((1, DIM)), full((1, DIM)),
        ],
        out_specs=pl.BlockSpec((1, N, DIM), lambda b: (b, 0, 0)),
        out_shape=jax.ShapeDtypeStruct((B, N, DIM), jnp.float32),
    )(
        feats,
        params['ffn_W1'], row2(params['ffn_b1']),
        params['ffn_W2'], row2(params['ffn_b2']),
        row2(params['prelu_a']),
        row2(params['norm1_g']), row2(params['norm1_b']),
        row2(params['norm2_g']), row2(params['norm2_b']),
    )


@jax.jit
def kernel(x, z, params):
    feats = _embed(z, params['token_emb'], params['pos_emb'][:N])
    coors = x
    for lp in params['layers']:
        feats, coors = _layer(feats, coors, lp)
    h = _head(feats, params)
    return h, coors


# speedup vs baseline: 6.2222x; 1.1049x over previous
import functools

import jax
import jax.numpy as jnp
from jax import lax
from jax.experimental import pallas as pl
from jax.experimental.pallas import tpu as pltpu

B, N = 2, 1024
DIM, HIDDEN = 128, 4
KNN, M_DIM = 32, 16
EDGE_IN = 2 * DIM + 1
ROWS = 256


def _silu(t):
    return t * (1.0 / (1.0 + jnp.exp(-t)))


def _ln(x, g, b):
    mu = jnp.mean(x, axis=-1, keepdims=True)
    var = jnp.mean((x - mu) ** 2, axis=-1, keepdims=True)
    return (x - mu) / jnp.sqrt(var + 1e-5) * g + b


def _dot(a, b):
    return jnp.dot(a, b, preferred_element_type=jnp.float32)


def _embed_body(z_ref, tok_ref, pos_ref, out_ref):
    z = z_ref[0]
    num_tok = tok_ref.shape[0]
    cols = lax.broadcasted_iota(jnp.int32, (N, num_tok), 1)
    onehot = (z == cols).astype(jnp.float32)
    emb = jnp.dot(onehot, tok_ref[...], precision=lax.Precision.HIGHEST,
                  preferred_element_type=jnp.float32)
    out_ref[0] = emb + pos_ref[...]


def _embed(z, token_emb, pos_emb):
    z2 = z.reshape(B, N, 1).astype(jnp.int32)
    return pl.pallas_call(
        _embed_body,
        grid=(B,),
        in_specs=[
            pl.BlockSpec((1, N, 1), lambda b: (b, 0, 0)),
            pl.BlockSpec(token_emb.shape, lambda b: (0, 0)),
            pl.BlockSpec(pos_emb.shape, lambda b: (0, 0)),
        ],
        out_specs=pl.BlockSpec((1, N, DIM), lambda b: (b, 0, 0)),
        out_shape=jax.ShapeDtypeStruct((B, N, DIM), jnp.float32),
    )(z2, token_emb, pos_emb)


def _layer_body(
    feats_all_ref, feats_blk_ref, coorsT_ref, coors_blk_ref,
    w1a_ref, w1b_ref, w1d_ref, b1_ref, w2_ref, b2_ref,
    cw1_ref, cb1_ref, cw2_ref, cb2_ref, cns_ref,
    nng_ref, nnb_ref, nw1a_ref, nw1b_ref, nb1_ref, nw2_ref, nb2_ref,
    feats_out_ref, coors_out_ref, d_ref,
):
    feats_all = feats_all_ref[0]
    feats_blk = feats_blk_ref[0]
    coors_blk = coors_blk_ref[0]

    d = None
    for c in range(3):
        xi_c = coors_blk[:, c:c + 1]
        xj_c = coorsT_ref[0, c:c + 1, :]
        rel_c = xi_c - xj_c
        sq = rel_c * rel_c
        d = sq if d is None else d + sq

    cols = lax.broadcasted_iota(jnp.int32, (ROWS, N), 1)
    d_ref[...] = (lax.bitcast_convert_type(d, jnp.int32) & ~(N - 1)) | cols

    gi = _dot(feats_blk, w1a_ref[...]) + b1_ref[...]
    feats_all_b = feats_all.astype(jnp.bfloat16)

    def body(k, carry):
        m_i, cd0_, cd1_, cd2_ = carry
        dcur = d_ref[...]
        mkey = jnp.min(dcur, axis=1, keepdims=True)
        hot = dcur == mkey
        d_ref[...] = jnp.where(hot, jnp.iinfo(jnp.int32).max, dcur)
        mval = lax.bitcast_convert_type(mkey & ~(N - 1), jnp.float32)
        onehot = hot.astype(jnp.bfloat16)

        fj = _dot(onehot, feats_all_b)
        xjs = [jnp.sum(jnp.where(hot, coorsT_ref[0, c:c + 1, :], 0.0),
                       axis=1, keepdims=True) for c in range(3)]
        relk = [coors_blk[:, c:c + 1] - xjs[c] for c in range(3)]

        pre = _dot(fj, w1b_ref[...]) + gi + mval * w1d_ref[...]
        h = _silu(pre)
        mk = _silu(_dot(h, w2_ref[...]) + b2_ref[...])
        cwh = _silu(_dot(mk, cw1_ref[...]) + cb1_ref[...])
        cw = _dot(cwh, cw2_ref[...]) + cb2_ref[...]

        scale = cw / jnp.clip(jnp.sqrt(mval), 1e-8) * cns_ref[0, 0]
        return (m_i + mk, cd0_ + scale * relk[0],
                cd1_ + scale * relk[1], cd2_ + scale * relk[2])

    zero1 = jnp.zeros((ROWS, 1), jnp.float32)
    m_i, cd0, cd1, cd2 = lax.fori_loop(
        0, KNN, body, (jnp.zeros((ROWS, M_DIM), jnp.float32), zero1, zero1, zero1))
    cdelta = jnp.concatenate([cd0, cd1, cd2], axis=1)

    nf = _ln(feats_blk, nng_ref[...], nnb_ref[...])
    pre2 = _dot(nf, nw1a_ref[...]) + _dot(m_i, nw1b_ref[...]) + nb1_ref[...]
    node_out = _dot(_silu(pre2), nw2_ref[...]) + nb2_ref[...] + feats_blk

    feats_out_ref[0] = node_out
    coors_out_ref[0] = coors_blk + cdelta


def _layer(feats, coors, lp):
    coorsT = jnp.swapaxes(coors, 1, 2)
    w1a = lp['e_W1'][:DIM]
    w1b = lp['e_W1'][DIM:2 * DIM]
    w1d = lp['e_W1'][2 * DIM:2 * DIM + 1]
    nw1a = lp['n_W1'][:DIM]
    nw1b = lp['n_W1'][DIM:]
    row2 = lambda v: v.reshape(1, -1)

    grid = (B, N // ROWS)
    full = lambda shape: pl.BlockSpec(shape, lambda b, r: (0,) * len(shape))
    perb = lambda shape: pl.BlockSpec((1,) + shape, lambda b, r: (b,) + (0,) * len(shape))
    blk = lambda shape: pl.BlockSpec((1, ROWS) + shape, lambda b, r: (b, r) + (0,) * len(shape))

    out = pl.pallas_call(
        _layer_body,
        grid=grid,
        in_specs=[
            perb((N, DIM)),
            blk((DIM,)),
            perb((3, N)),
            blk((3,)),
            full(w1a.shape), full(w1b.shape), full((1, 2 * EDGE_IN)),
            full((1, 2 * EDGE_IN)),
            full(lp['e_W2'].shape), full((1, M_DIM)),
            full(lp['c_W1'].shape), full((1, 4 * M_DIM)),
            full(lp['c_W2'].shape), full((1, 1)), full((1, 1)),
            full((1, DIM)), full((1, DIM)),
            full(nw1a.shape), full(nw1b.shape), full((1, 2 * DIM)),
            full(lp['n_W2'].shape), full((1, DIM)),
        ],
        out_specs=[blk((DIM,)), blk((3,))],
        out_shape=[
            jax.ShapeDtypeStruct((B, N, DIM), jnp.float32),
            jax.ShapeDtypeStruct((B, N, 3), jnp.float32),
        ],
        scratch_shapes=[pltpu.VMEM((ROWS, N), jnp.int32)],
    )(
        feats, feats, coorsT, coors,
        w1a, w1b, row2(w1d), row2(lp['e_b1']), lp['e_W2'], row2(lp['e_b2']),
        lp['c_W1'], row2(lp['c_b1']), lp['c_W2'], row2(lp['c_b2']),
        row2(lp['cn_scale']),
        row2(lp['nn_g']), row2(lp['nn_b']),
        nw1a, nw1b, row2(lp['n_b1']), lp['n_W2'], row2(lp['n_b2']),
    )
    return out[0], out[1]


def _head_body(x_ref, w1_ref, b1_ref, w2_ref, b2_ref, pa_ref,
               g1_ref, bb1_ref, g2_ref, bb2_ref, out_ref):
    x = x_ref[0]
    h = _ln(x + x, g1_ref[...], bb1_ref[...])
    a = _dot(h, w1_ref[...]) + b1_ref[...]
    a = jnp.where(a >= 0, a, pa_ref[0, 0] * a)
    h2 = _dot(a, w2_ref[...]) + b2_ref[...]
    out_ref[0] = _ln(h + h2, g2_ref[...], bb2_ref[...])


def _head(feats, params):
    row2 = lambda v: v.reshape(1, -1)
    full = lambda shape: pl.BlockSpec(shape, lambda b: (0,) * len(shape))
    return pl.pallas_call(
        _head_body,
        grid=(B,),
        in_specs=[
            pl.BlockSpec((1, N, DIM), lambda b: (b, 0, 0)),
            full(params['ffn_W1'].shape), full((1, HIDDEN * DIM)),
            full(params['ffn_W2'].shape), full((1, DIM)),
            full((1, 1)),
            full((1, DIM)), full((1, DIM)), full((1, DIM)), full((1, DIM)),
        ],
        out_specs=pl.BlockSpec((1, N, DIM), lambda b: (b, 0, 0)),
        out_shape=jax.ShapeDtypeStruct((B, N, DIM), jnp.float32),
    )(
        feats,
        params['ffn_W1'], row2(params['ffn_b1']),
        params['ffn_W2'], row2(params['ffn_b2']),
        row2(params['prelu_a']),
        row2(params['norm1_g']), row2(params['norm1_b']),
        row2(params['norm2_g']), row2(params['norm2_b']),
    )


@jax.jit
def kernel(x, z, params):
    feats = _embed(z, params['token_emb'], params['pos_emb'][:N])
    coors = x
    for lp in params['layers']:
        feats, coors = _layer(feats, coors, lp)
    h = _head(feats, params)
    return h, coors
